# Initial kernel scaffold; baseline (speedup 1.0000x reference)
import jax
import jax.numpy as jnp
from jax.experimental import pallas as pl
from jax.experimental.pallas import tpu as pltpu

_B, _S, _H = 4, 8192, 768
_E, _TOPK = 8, 2
_N = _B * _S

_BLOCK_T = 2048
_GRID = _N // _BLOCK_T


def _router_kernel(x_ref, w_ref, probs_ref, wts_ref, idx_ref, loss_ref,
                   ew_ref, ec_ref):
    i = pl.program_id(0)

    @pl.when(i == 0)
    def _init():
        ew_ref[...] = jnp.zeros_like(ew_ref)
        ec_ref[...] = jnp.zeros_like(ec_ref)

    x = x_ref[...]
    w = w_ref[...]
    logits = jax.lax.dot_general(
        x, w, (((1,), (1,)), ((), ())),
        preferred_element_type=jnp.float32)

    m = jnp.max(logits, axis=-1, keepdims=True)
    ex = jnp.exp(logits - m)
    probs = ex / jnp.sum(ex, axis=-1, keepdims=True)
    probs_ref[...] = probs

    iota = jax.lax.broadcasted_iota(jnp.int32, probs.shape, 1)
    m1 = jnp.max(probs, axis=-1, keepdims=True)
    i1 = jnp.min(jnp.where(probs == m1, iota, _E), axis=-1, keepdims=True)
    probs2 = jnp.where(iota == i1, -jnp.inf, probs)
    m2 = jnp.max(probs2, axis=-1, keepdims=True)
    i2 = jnp.min(jnp.where(probs2 == m2, iota, _E), axis=-1, keepdims=True)

    denom = m1 + m2 + 1e-8
    w1 = m1 / denom
    w2 = m2 / denom

    col = jax.lax.broadcasted_iota(jnp.int32, (_BLOCK_T, _TOPK), 1)
    wts_ref[...] = jnp.where(col == 0, w1, w2)
    idx_ref[...] = jnp.where(col == 0, i1, i2)

    onehot1 = (iota == i1).astype(jnp.float32)
    onehot2 = (iota == i2).astype(jnp.float32)
    ew_ref[...] += jnp.sum(onehot1 * w1 + onehot2 * w2, axis=0,
                           keepdims=True)
    ec_ref[...] += jnp.sum(onehot1 + onehot2, axis=0, keepdims=True)

    @pl.when(i == _GRID - 1)
    def _fini():
        expected = _N * _TOPK / _E
        loss_ref[0, 0] = jnp.sum(ew_ref[...] * ec_ref[...]) / (expected *
                                                               expected)


def kernel(hidden_states, W):
    x = hidden_states.reshape(_N, _H)
    probs, wts, idx, loss = pl.pallas_call(
        _router_kernel,
        grid=(_GRID,),
        in_specs=[
            pl.BlockSpec((_BLOCK_T, _H), lambda i: (i, 0)),
            pl.BlockSpec((_E, _H), lambda i: (0, 0)),
        ],
        out_specs=[
            pl.BlockSpec((_BLOCK_T, _E), lambda i: (i, 0)),
            pl.BlockSpec((_BLOCK_T, _TOPK), lambda i: (i, 0)),
            pl.BlockSpec((_BLOCK_T, _TOPK), lambda i: (i, 0)),
            pl.BlockSpec((1, 1), lambda i: (0, 0)),
        ],
        out_shape=[
            jax.ShapeDtypeStruct((_N, _E), jnp.float32),
            jax.ShapeDtypeStruct((_N, _TOPK), jnp.float32),
            jax.ShapeDtypeStruct((_N, _TOPK), jnp.int32),
            jax.ShapeDtypeStruct((1, 1), jnp.float32),
        ],
        scratch_shapes=[
            pltpu.VMEM((1, _E), jnp.float32),
            pltpu.VMEM((1, _E), jnp.float32),
        ],
        compiler_params=pltpu.CompilerParams(
            dimension_semantics=("arbitrary",)),
    )(x, W)
    return (wts.reshape(_B, _S, _TOPK), idx.reshape(_B, _S, _TOPK),
            probs.reshape(_B, _S, _E), loss[0, 0])


# speedup vs baseline: 2.3431x; 2.3431x over previous
import jax
import jax.numpy as jnp
from jax.experimental import pallas as pl
from jax.experimental.pallas import tpu as pltpu

_B, _S, _H = 4, 8192, 768
_E, _TOPK = 8, 2
_N = _B * _S

_BLOCK_T = 2048
_GRID = _N // _BLOCK_T


def _router_kernel(x_ref, w_ref, probs_ref, wts_ref, idx_ref, loss_ref,
                   ew_ref, ec_ref):
    i = pl.program_id(0)

    @pl.when(i == 0)
    def _init():
        ew_ref[...] = jnp.zeros_like(ew_ref)
        ec_ref[...] = jnp.zeros_like(ec_ref)

    x = x_ref[...]
    w = w_ref[...]
    logits = jax.lax.dot_general(
        x, w, (((1,), (1,)), ((), ())),
        preferred_element_type=jnp.float32)

    m = jnp.max(logits, axis=-1, keepdims=True)
    ex = jnp.exp(logits - m)
    probs = ex / jnp.sum(ex, axis=-1, keepdims=True)
    probs_ref[...] = probs

    iota = jax.lax.broadcasted_iota(jnp.int32, probs.shape, 1)
    m1 = jnp.max(probs, axis=-1, keepdims=True)
    i1 = jnp.min(jnp.where(probs == m1, iota, _E), axis=-1, keepdims=True)
    probs2 = jnp.where(iota == i1, -jnp.inf, probs)
    m2 = jnp.max(probs2, axis=-1, keepdims=True)
    i2 = jnp.min(jnp.where(probs2 == m2, iota, _E), axis=-1, keepdims=True)

    denom = m1 + m2 + 1e-8
    w1 = m1 / denom
    w2 = m2 / denom

    col = jax.lax.broadcasted_iota(jnp.int32, (_BLOCK_T, _TOPK), 1)
    wts_ref[...] = jnp.where(col == 0, w1, w2)
    idx_ref[...] = jnp.where(col == 0, i1, i2)

    onehot1 = (iota == i1).astype(jnp.float32)
    onehot2 = (iota == i2).astype(jnp.float32)
    ew_ref[...] += jnp.sum(onehot1 * w1 + onehot2 * w2, axis=0,
                           keepdims=True)
    ec_ref[...] += jnp.sum(onehot1 + onehot2, axis=0, keepdims=True)

    @pl.when(i == _GRID - 1)
    def _fini():
        expected = _N * _TOPK / _E
        loss_ref[...] = jnp.sum(ew_ref[...] * ec_ref[...], axis=1,
                                keepdims=True) / (expected * expected)


def kernel(hidden_states, W):
    x = hidden_states.reshape(_N, _H)
    probs, wts, idx, loss = pl.pallas_call(
        _router_kernel,
        grid=(_GRID,),
        in_specs=[
            pl.BlockSpec((_BLOCK_T, _H), lambda i: (i, 0)),
            pl.BlockSpec((_E, _H), lambda i: (0, 0)),
        ],
        out_specs=[
            pl.BlockSpec((_BLOCK_T, _E), lambda i: (i, 0)),
            pl.BlockSpec((_BLOCK_T, _TOPK), lambda i: (i, 0)),
            pl.BlockSpec((_BLOCK_T, _TOPK), lambda i: (i, 0)),
            pl.BlockSpec((1, 1), lambda i: (0, 0)),
        ],
        out_shape=[
            jax.ShapeDtypeStruct((_N, _E), jnp.float32),
            jax.ShapeDtypeStruct((_N, _TOPK), jnp.float32),
            jax.ShapeDtypeStruct((_N, _TOPK), jnp.int32),
            jax.ShapeDtypeStruct((1, 1), jnp.float32),
        ],
        scratch_shapes=[
            pltpu.VMEM((1, _E), jnp.float32),
            pltpu.VMEM((1, _E), jnp.float32),
        ],
        compiler_params=pltpu.CompilerParams(
            dimension_semantics=("arbitrary",)),
    )(x, W)
    return (wts.reshape(_B, _S, _TOPK), idx.reshape(_B, _S, _TOPK),
            probs.reshape(_B, _S, _E), loss[0, 0])


# speedup vs baseline: 6.0583x; 2.5856x over previous
import jax
import jax.numpy as jnp
from jax.experimental import pallas as pl
from jax.experimental.pallas import tpu as pltpu

_B, _S, _H = 4, 8192, 768
_E, _TOPK = 8, 2
_N = _B * _S

_BLOCK_T = 2048
_GRID = _N // _BLOCK_T


def _router_kernel(x_ref, w_ref, probs_ref, wts_ref, idx_ref, loss_ref,
                   ew_ref, ec_ref):
    i = pl.program_id(0)

    @pl.when(i == 0)
    def _init():
        ew_ref[...] = jnp.zeros_like(ew_ref)
        ec_ref[...] = jnp.zeros_like(ec_ref)

    x = x_ref[...]
    w = w_ref[...]
    logits = jax.lax.dot_general(
        w, x, (((1,), (1,)), ((), ())),
        preferred_element_type=jnp.float32)

    m = jnp.max(logits, axis=0, keepdims=True)
    ex = jnp.exp(logits - m)
    probs = ex * (1.0 / jnp.sum(ex, axis=0, keepdims=True))
    probs_ref[...] = probs

    iota = jax.lax.broadcasted_iota(jnp.int32, probs.shape, 0)
    m1 = jnp.max(probs, axis=0, keepdims=True)
    i1 = jnp.min(jnp.where(probs == m1, iota, _E), axis=0, keepdims=True)
    probs2 = jnp.where(iota == i1, -1.0, probs)
    m2 = jnp.max(probs2, axis=0, keepdims=True)
    i2 = jnp.min(jnp.where(probs2 == m2, iota, _E), axis=0, keepdims=True)

    denom = m1 + m2 + 1e-8
    w1 = m1 / denom
    w2 = m2 / denom

    row = jax.lax.broadcasted_iota(jnp.int32, (_TOPK, _BLOCK_T), 0)
    wts_ref[...] = jnp.where(row == 0, w1, w2)
    idx_ref[...] = jnp.where(row == 0, i1, i2)

    onehot1 = (iota == i1).astype(jnp.float32)
    onehot2 = (iota == i2).astype(jnp.float32)
    cw = (onehot1 * w1 + onehot2 * w2).reshape(_E, _BLOCK_T // 128, 128)
    cc = (onehot1 + onehot2).reshape(_E, _BLOCK_T // 128, 128)
    ew_ref[...] += jnp.sum(cw, axis=1)
    ec_ref[...] += jnp.sum(cc, axis=1)

    @pl.when(i == _GRID - 1)
    def _fini():
        expected = _N * _TOPK / _E
        ew = jnp.sum(ew_ref[...], axis=1, keepdims=True)
        ec = jnp.sum(ec_ref[...], axis=1, keepdims=True)
        loss_ref[...] = jnp.sum(ew * ec, axis=0, keepdims=True) / (
            expected * expected)


def kernel(hidden_states, W):
    x = hidden_states.reshape(_N, _H)
    probs_t, wts_t, idx_t, loss = pl.pallas_call(
        _router_kernel,
        grid=(_GRID,),
        in_specs=[
            pl.BlockSpec((_BLOCK_T, _H), lambda i: (i, 0)),
            pl.BlockSpec((_E, _H), lambda i: (0, 0)),
        ],
        out_specs=[
            pl.BlockSpec((_E, _BLOCK_T), lambda i: (0, i)),
            pl.BlockSpec((_TOPK, _BLOCK_T), lambda i: (0, i)),
            pl.BlockSpec((_TOPK, _BLOCK_T), lambda i: (0, i)),
            pl.BlockSpec((1, 1), lambda i: (0, 0)),
        ],
        out_shape=[
            jax.ShapeDtypeStruct((_E, _N), jnp.float32),
            jax.ShapeDtypeStruct((_TOPK, _N), jnp.float32),
            jax.ShapeDtypeStruct((_TOPK, _N), jnp.int32),
            jax.ShapeDtypeStruct((1, 1), jnp.float32),
        ],
        scratch_shapes=[
            pltpu.VMEM((_E, 128), jnp.float32),
            pltpu.VMEM((_E, 128), jnp.float32),
        ],
        compiler_params=pltpu.CompilerParams(
            dimension_semantics=("arbitrary",)),
    )(x, W)
    return (wts_t.T.reshape(_B, _S, _TOPK), idx_t.T.reshape(_B, _S, _TOPK),
            probs_t.T.reshape(_B, _S, _E), loss[0, 0])
